# baseline (speedup 1.0000x reference)
import functools

import jax
import jax.numpy as jnp
from jax import lax
from jax.experimental import pallas as pl
from jax.experimental.pallas import tpu as pltpu
from jax.experimental.pallas import tpu_sc as plsc

VOCAB = 100000
EMBED_DIM = 16
BATCH = 1024
CTX = 20

_INFO = plsc.get_sparse_core_info()
_NC, _NS = _INFO.num_cores, _INFO.num_subcores
_NW = _NC * _NS
_BPW = BATCH // _NW


def _make_gather_mean():
    mesh = plsc.VectorSubcoreMesh(core_axis_name="c", subcore_axis_name="s")

    @functools.partial(
        pl.kernel,
        mesh=mesh,
        out_type=jax.ShapeDtypeStruct((BATCH, EMBED_DIM), jnp.float32),
        scratch_types=[
            pltpu.VMEM((_BPW * CTX,), jnp.int32),
            pltpu.VMEM((_BPW * CTX, EMBED_DIM), jnp.float32),
            pltpu.VMEM((_BPW, EMBED_DIM), jnp.float32),
            pltpu.SemaphoreType.DMA,
        ],
        compiler_params=pltpu.CompilerParams(use_tc_tiling_on_sc=False),
    )
    def gather_mean(ctx_hbm, table_hbm, out_hbm, idx_v, rows_v, out_v, sem):
        wid = lax.axis_index("s") * _NC + lax.axis_index("c")
        base = wid * _BPW
        pltpu.sync_copy(ctx_hbm.at[pl.ds(base * CTX, _BPW * CTX)], idx_v)
        pltpu.async_copy(table_hbm.at[idx_v], rows_v, sem).wait()

        def body(b, _):
            acc = rows_v[b * CTX]
            for j in range(1, CTX):
                acc = acc + rows_v[b * CTX + j]
            out_v[b] = acc * (1.0 / CTX)
            return 0

        lax.fori_loop(0, _BPW, body, 0)
        pltpu.sync_copy(out_v, out_hbm.at[pl.ds(base, _BPW)])

    return gather_mean


_gather_mean = _make_gather_mean()

_VT = 1024
_NVT = (VOCAB + _VT - 1) // _VT


def _mm_body(wt_ref, avg_ref, b_ref, outt_ref):
    wtb = jnp.concatenate([wt_ref[...], b_ref[...]], axis=0)
    avg1 = jnp.concatenate(
        [avg_ref[...], jnp.ones((BATCH, 1), jnp.float32)], axis=1
    )
    outt_ref[...] = lax.dot_general(
        wtb,
        avg1,
        (((0,), (1,)), ((), ())),
        preferred_element_type=jnp.float32,
        precision=lax.Precision.DEFAULT,
    )


def _projection(avg, Wt, bcol):
    return pl.pallas_call(
        _mm_body,
        grid=(_NVT,),
        in_specs=[
            pl.BlockSpec((EMBED_DIM, _VT), lambda i: (0, i)),
            pl.BlockSpec((BATCH, EMBED_DIM), lambda i: (0, 0)),
            pl.BlockSpec((1, _VT), lambda i: (0, i)),
        ],
        out_specs=pl.BlockSpec((_VT, BATCH), lambda i: (i, 0)),
        out_shape=jax.ShapeDtypeStruct((VOCAB, BATCH), jnp.float32),
        compiler_params=pltpu.CompilerParams(
            dimension_semantics=("arbitrary",),
        ),
    )(Wt, avg, bcol)


@jax.jit
def kernel(context_words, in_emb, W, b):
    ctx_flat = context_words.reshape(-1).astype(jnp.int32)
    avg = _gather_mean(ctx_flat, in_emb)
    logits_t = _projection(avg, W.T, b.reshape(1, VOCAB))
    return logits_t.T


# speedup vs baseline: 1.0701x; 1.0701x over previous
import functools

import jax
import jax.numpy as jnp
from jax import lax
from jax.experimental import pallas as pl
from jax.experimental.pallas import tpu as pltpu
from jax.experimental.pallas import tpu_sc as plsc

VOCAB = 100000
EMBED_DIM = 16
BATCH = 1024
CTX = 20

_INFO = plsc.get_sparse_core_info()
_NC, _NS = _INFO.num_cores, _INFO.num_subcores
_NW = _NC * _NS
_BPW = BATCH // _NW


def _make_gather_mean():
    mesh = plsc.VectorSubcoreMesh(core_axis_name="c", subcore_axis_name="s")

    @functools.partial(
        pl.kernel,
        mesh=mesh,
        out_type=jax.ShapeDtypeStruct((BATCH, EMBED_DIM), jnp.float32),
        scratch_types=[
            pltpu.VMEM((_BPW * CTX,), jnp.int32),
            pltpu.VMEM((_BPW * CTX, EMBED_DIM), jnp.float32),
            pltpu.VMEM((_BPW, EMBED_DIM), jnp.float32),
            pltpu.SemaphoreType.DMA,
        ],
        compiler_params=pltpu.CompilerParams(use_tc_tiling_on_sc=False),
    )
    def gather_mean(ctx_hbm, table_hbm, out_hbm, idx_v, rows_v, out_v, sem):
        wid = lax.axis_index("s") * _NC + lax.axis_index("c")
        base = wid * _BPW
        pltpu.sync_copy(ctx_hbm.at[pl.ds(base * CTX, _BPW * CTX)], idx_v)
        pltpu.async_copy(table_hbm.at[idx_v], rows_v, sem).wait()

        def body(b, _):
            acc = rows_v[b * CTX]
            for j in range(1, CTX):
                acc = acc + rows_v[b * CTX + j]
            out_v[b] = acc * (1.0 / CTX)
            return 0

        lax.fori_loop(0, _BPW, body, 0)
        pltpu.sync_copy(out_v, out_hbm.at[pl.ds(base, _BPW)])

    return gather_mean


_gather_mean = _make_gather_mean()

_VT = 3072
_NVT = (VOCAB + _VT - 1) // _VT


def _mm_body(wt_ref, avg_ref, b_ref, outt_ref):
    wtb = jnp.concatenate([wt_ref[...], b_ref[...]], axis=0)
    avg1 = jnp.concatenate(
        [avg_ref[...], jnp.ones((BATCH, 1), jnp.float32)], axis=1
    )
    outt_ref[...] = lax.dot_general(
        wtb,
        avg1,
        (((0,), (1,)), ((), ())),
        preferred_element_type=jnp.float32,
        precision=lax.Precision.DEFAULT,
    )


def _projection(avg, Wt, bcol):
    return pl.pallas_call(
        _mm_body,
        grid=(_NVT,),
        in_specs=[
            pl.BlockSpec((EMBED_DIM, _VT), lambda i: (0, i)),
            pl.BlockSpec((BATCH, EMBED_DIM), lambda i: (0, 0)),
            pl.BlockSpec((1, _VT), lambda i: (0, i)),
        ],
        out_specs=pl.BlockSpec((_VT, BATCH), lambda i: (i, 0)),
        out_shape=jax.ShapeDtypeStruct((VOCAB, BATCH), jnp.float32),
        compiler_params=pltpu.CompilerParams(
            dimension_semantics=("arbitrary",),
        ),
    )(Wt, avg, bcol)


@jax.jit
def kernel(context_words, in_emb, W, b):
    ctx_flat = context_words.reshape(-1).astype(jnp.int32)
    avg = _gather_mean(ctx_flat, in_emb)
    logits_t = _projection(avg, W.T, b.reshape(1, VOCAB))
    return logits_t.T


# speedup vs baseline: 1.1058x; 1.0333x over previous
import functools

import jax
import jax.numpy as jnp
from jax import lax
from jax.experimental import pallas as pl
from jax.experimental.pallas import tpu as pltpu
from jax.experimental.pallas import tpu_sc as plsc

VOCAB = 100000
EMBED_DIM = 16
BATCH = 1024
CTX = 20

_INFO = plsc.get_sparse_core_info()
_NC, _NS = _INFO.num_cores, _INFO.num_subcores
_NW = _NC * _NS
_BPW = BATCH // _NW


_EPW = _BPW * CTX * EMBED_DIM


def _make_gather_mean():
    mesh = plsc.VectorSubcoreMesh(core_axis_name="c", subcore_axis_name="s")

    @functools.partial(
        pl.kernel,
        mesh=mesh,
        out_type=jax.ShapeDtypeStruct((BATCH, EMBED_DIM), jnp.float32),
        scratch_types=[
            pltpu.VMEM((_EPW,), jnp.int32),
            pltpu.VMEM((_EPW,), jnp.float32),
            pltpu.VMEM((_BPW, EMBED_DIM), jnp.float32),
            pltpu.SemaphoreType.DMA,
        ],
        compiler_params=pltpu.CompilerParams(use_tc_tiling_on_sc=False),
    )
    def gather_mean(idx_hbm, tablet_hbm, out_hbm, idx_v, rows_v, out_v, sem):
        wid = lax.axis_index("s") * _NC + lax.axis_index("c")
        base = wid * _BPW
        pltpu.sync_copy(idx_hbm.at[pl.ds(wid * _EPW, _EPW)], idx_v)
        pltpu.async_copy(tablet_hbm.at[idx_v], rows_v, sem).wait()

        def body(b, _):
            acc = rows_v[pl.ds(b * (CTX * EMBED_DIM), EMBED_DIM)]
            for j in range(1, CTX):
                acc = acc + rows_v[
                    pl.ds(b * (CTX * EMBED_DIM) + j * EMBED_DIM, EMBED_DIM)
                ]
            out_v[b] = acc * (1.0 / CTX)
            return 0

        lax.fori_loop(0, _BPW, body, 0)
        pltpu.sync_copy(out_v, out_hbm.at[pl.ds(base, _BPW)])

    return gather_mean


_gather_mean = _make_gather_mean()

_VT = 2048
_NVT = (VOCAB + _VT - 1) // _VT


def _mm_body(wt_ref, avg_ref, b_ref, outt_ref):
    wtb = jnp.concatenate([wt_ref[...], b_ref[...]], axis=0)
    avg1 = jnp.concatenate(
        [avg_ref[...], jnp.ones((BATCH, 1), jnp.float32)], axis=1
    )
    outt_ref[...] = lax.dot_general(
        wtb,
        avg1,
        (((0,), (1,)), ((), ())),
        preferred_element_type=jnp.float32,
        precision=lax.Precision.DEFAULT,
    )


def _projection(avg, Wt, bcol):
    return pl.pallas_call(
        _mm_body,
        grid=(_NVT,),
        in_specs=[
            pl.BlockSpec((EMBED_DIM, _VT), lambda i: (0, i)),
            pl.BlockSpec((BATCH, EMBED_DIM), lambda i: (0, 0)),
            pl.BlockSpec((1, _VT), lambda i: (0, i)),
        ],
        out_specs=pl.BlockSpec((_VT, BATCH), lambda i: (i, 0)),
        out_shape=jax.ShapeDtypeStruct((VOCAB, BATCH), jnp.float32),
        compiler_params=pltpu.CompilerParams(
            dimension_semantics=("arbitrary",),
        ),
    )(Wt, avg, bcol)


@jax.jit
def kernel(context_words, in_emb, W, b):
    ctx_flat = context_words.reshape(-1).astype(jnp.int32)
    dvec = jnp.arange(EMBED_DIM, dtype=jnp.int32) * VOCAB
    idx_flat = (ctx_flat[:, None] + dvec[None, :]).reshape(-1)
    tablet_flat = in_emb.T.reshape(-1)
    avg = _gather_mean(idx_flat, tablet_flat)
    logits_t = _projection(avg, W.T, b.reshape(1, VOCAB))
    return logits_t.T


# speedup vs baseline: 1.2249x; 1.1077x over previous
import functools

import jax
import jax.numpy as jnp
from jax import lax
from jax.experimental import pallas as pl
from jax.experimental.pallas import tpu as pltpu
from jax.experimental.pallas import tpu_sc as plsc

VOCAB = 100000
EMBED_DIM = 16
BATCH = 1024
CTX = 20

_INFO = plsc.get_sparse_core_info()
_NC, _NS = _INFO.num_cores, _INFO.num_subcores
_NW = _NC * _NS
_BPW = BATCH // _NW


_EPW = _BPW * CTX * EMBED_DIM


def _make_gather_mean():
    mesh = plsc.VectorSubcoreMesh(core_axis_name="c", subcore_axis_name="s")

    @functools.partial(
        pl.kernel,
        mesh=mesh,
        out_type=jax.ShapeDtypeStruct((BATCH, EMBED_DIM), jnp.float32),
        scratch_types=[
            pltpu.VMEM((_BPW * CTX,), jnp.int32),
            pltpu.VMEM((_EPW,), jnp.int32),
            pltpu.VMEM((_EPW,), jnp.float32),
            pltpu.VMEM((_BPW, EMBED_DIM), jnp.float32),
            pltpu.SemaphoreType.DMA,
        ],
        compiler_params=pltpu.CompilerParams(
            use_tc_tiling_on_sc=False, needs_layout_passes=False
        ),
    )
    def gather_mean(ctx_hbm, tablet_hbm, out_hbm, ctx_v, idx_v, rows_v, out_v, sem):
        wid = lax.axis_index("s") * _NC + lax.axis_index("c")
        base = wid * _BPW
        pltpu.sync_copy(ctx_hbm.at[pl.ds(base * CTX, _BPW * CTX)], ctx_v)

        def build(kc, _):
            kvec = (lax.iota(jnp.int32, 16) + kc * 16) * EMBED_DIM
            c16 = ctx_v[pl.ds(kc * 16, 16)]
            for d in range(EMBED_DIM):
                plsc.store_scatter(idx_v, [kvec + d], c16 + d * VOCAB)
            return 0

        lax.fori_loop(0, (_BPW * CTX) // 16, build, 0)
        pltpu.async_copy(tablet_hbm.at[idx_v], rows_v, sem).wait()

        def body(b, _):
            acc = rows_v[pl.ds(b * (CTX * EMBED_DIM), EMBED_DIM)]
            for j in range(1, CTX):
                acc = acc + rows_v[
                    pl.ds(b * (CTX * EMBED_DIM) + j * EMBED_DIM, EMBED_DIM)
                ]
            out_v[b] = acc * (1.0 / CTX)
            return 0

        lax.fori_loop(0, _BPW, body, 0)
        pltpu.sync_copy(out_v, out_hbm.at[pl.ds(base, _BPW)])

    return gather_mean


_gather_mean = _make_gather_mean()

_VT = 2048
_NVT = (VOCAB + _VT - 1) // _VT


def _mm_body(wt_ref, avg_ref, b_ref, outt_ref):
    wtb = jnp.concatenate([wt_ref[...], b_ref[...]], axis=0)
    avg1 = jnp.concatenate(
        [avg_ref[...], jnp.ones((BATCH, 1), jnp.float32)], axis=1
    )
    outt_ref[...] = lax.dot_general(
        wtb,
        avg1,
        (((0,), (1,)), ((), ())),
        preferred_element_type=jnp.float32,
        precision=lax.Precision.DEFAULT,
    )


def _projection(avg, Wt, bcol):
    return pl.pallas_call(
        _mm_body,
        grid=(_NVT,),
        in_specs=[
            pl.BlockSpec((EMBED_DIM, _VT), lambda i: (0, i)),
            pl.BlockSpec((BATCH, EMBED_DIM), lambda i: (0, 0)),
            pl.BlockSpec((1, _VT), lambda i: (0, i)),
        ],
        out_specs=pl.BlockSpec((_VT, BATCH), lambda i: (i, 0)),
        out_shape=jax.ShapeDtypeStruct((VOCAB, BATCH), jnp.float32),
        compiler_params=pltpu.CompilerParams(
            dimension_semantics=("arbitrary",),
        ),
    )(Wt, avg, bcol)


@jax.jit
def kernel(context_words, in_emb, W, b):
    ctx_flat = context_words.reshape(-1).astype(jnp.int32)
    tablet_flat = in_emb.T.reshape(-1)
    avg = _gather_mean(ctx_flat, tablet_flat)
    logits_t = _projection(avg, W.T, b.reshape(1, VOCAB))
    return logits_t.T
